# baseline (speedup 1.0000x reference)
import dataclasses
import functools

import jax
import jax.numpy as jnp
from jax import lax
from jax.experimental import pallas as pl
from jax.experimental.pallas import tpu as pltpu
from jax.experimental.pallas import tpu_sc as plsc

N = 10000
D = 128
G = 64
NC = 2
NS = 16
LANES = 16
NW = NC * NS
CHUNK = 64
ROWS_PER_TILE = 632
NACC = ROWS_PER_TILE * NS
_ZOFF = tuple(range(0, 513, 64)) + (568,)

_HI = lax.Precision.HIGHEST


def _cdiv(a, b):
    return (a + b - 1) // b


def _edge_body(nchunks, xl_hbm, xr_hbm, att_hbm, src_hbm, dst_hbm,
               numer_hbm, denom_hbm,
               numer_sh, denom_sh,
               src0, dst0, xl0, xr0, p0, src1, dst1, xl1, xr1, p1, att_v,
               gl0, gr0, sn0, sd0, gl1, gr1, sn1, sd1):
    c = lax.axis_index("c")
    s = lax.axis_index("s")
    w = c * NS + s
    bufs = ((src0, dst0, xl0, xr0, p0, gl0, gr0, sn0, sd0),
            (src1, dst1, xl1, xr1, p1, gl1, gr1, sn1, sd1))

    z16 = jnp.zeros((LANES,), jnp.float32)

    @pl.loop(0, CHUNK)
    def _(r):
        for j in range(D // LANES):
            xl0[r, pl.ds(j * LANES, LANES)] = z16

    @pl.loop(0, CHUNK, step=LANES)
    def _(i):
        p0[pl.ds(i, LANES)] = z16

    r0 = s * ROWS_PER_TILE
    for off in _ZOFF:
        pltpu.sync_copy(xl0, numer_sh.at[pl.ds(r0 + off, CHUNK)])
        pltpu.sync_copy(p0, denom_sh.at[pl.ds(r0 + off, CHUNK)])
    pltpu.sync_copy(att_hbm, att_v)
    plsc.subcore_barrier()

    def load(i, b):
        srcv, dstv, xlv, xrv, _, gl, gr, _, _ = bufs[b]
        base = (w * nchunks + i) * CHUNK
        pltpu.sync_copy(src_hbm.at[pl.ds(base, CHUNK)], srcv)
        pltpu.sync_copy(dst_hbm.at[pl.ds(base, CHUNK)], dstv)
        pltpu.async_copy(xl_hbm.at[srcv], xlv, gl)
        pltpu.async_copy(xr_hbm.at[dstv], xrv, gr)

    def wait_gathers(b):
        srcv, dstv, xlv, xrv, _, gl, gr, _, _ = bufs[b]
        pltpu.make_async_copy(xl_hbm.at[srcv], xlv, gl).wait()
        pltpu.make_async_copy(xr_hbm.at[dstv], xrv, gr).wait()

    att_regs = [att_v[pl.ds(j * LANES, LANES)] for j in range(D // LANES)]

    def compute(b):
        _, _, xlv, xrv, pv, _, _, _, _ = bufs[b]

        @pl.loop(0, CHUNK)
        def _(e):
            acc = jnp.zeros((LANES,), jnp.float32)
            for j in range(D // LANES):
                sl = pl.ds(j * LANES, LANES)
                u = xlv[e, sl] + xrv[e, sl]
                t = jnp.maximum(u, 0.2 * u)
                acc = acc + t * att_regs[j]
            xrv[e, pl.ds(0, LANES)] = acc

        @pl.loop(0, CHUNK, step=LANES)
        def _(eb):
            rows = eb + lax.iota(jnp.int32, LANES)
            ssum = jnp.zeros((LANES,), jnp.float32)
            for col in range(LANES):
                cols = jnp.full((LANES,), col, jnp.int32)
                ssum = ssum + plsc.load_gather(xrv, [rows, cols])
            p16 = jnp.exp(ssum)
            pv[pl.ds(eb, LANES)] = p16
            for l in range(LANES):
                p = p16[l]
                for j in range(D // LANES):
                    sl = pl.ds(j * LANES, LANES)
                    xlv[eb + l, sl] = xlv[eb + l, sl] * p

    def start_scatter(b):
        _, dstv, xlv, _, pv, _, _, sn, sd = bufs[b]
        pltpu.async_copy(xlv, numer_sh.at[dstv], sn, add=True)
        pltpu.async_copy(pv, denom_sh.at[dstv], sd, add=True)

    def wait_scatter(b):
        _, dstv, xlv, _, pv, _, _, sn, sd = bufs[b]
        pltpu.make_async_copy(xlv, numer_sh.at[dstv], sn).wait()
        pltpu.make_async_copy(pv, denom_sh.at[dstv], sd).wait()

    load(0, 0)

    @pl.loop(0, nchunks // 2)
    def _(g2):
        i0 = 2 * g2

        @pl.when(g2 > 0)
        def _():
            wait_scatter(1)

        load(i0 + 1, 1)
        wait_gathers(0)
        compute(0)
        start_scatter(0)
        wait_gathers(1)
        compute(1)
        start_scatter(1)
        wait_scatter(0)

        @pl.when(i0 + 2 < nchunks)
        def _():
            load(i0 + 2, 0)

    wait_scatter(1)

    plsc.subcore_barrier()
    pltpu.sync_copy(numer_sh.at[pl.ds(r0, ROWS_PER_TILE)],
                    numer_hbm.at[c, pl.ds(r0, ROWS_PER_TILE)])
    for off in _ZOFF:
        pltpu.sync_copy(denom_sh.at[pl.ds(r0 + off, CHUNK)], p0)
        pltpu.sync_copy(p0, denom_hbm.at[pl.ds(c * NACC + r0 + off, CHUNK)])


def _sc_edge_layer(xl, xr, att, src, dst, nchunks):
    mesh = plsc.VectorSubcoreMesh(core_axis_name="c", subcore_axis_name="s")
    cp = pltpu.CompilerParams()
    if "needs_layout_passes" in pltpu.CompilerParams.__dataclass_fields__:
        cp = dataclasses.replace(cp, needs_layout_passes=False)
    kfn = pl.kernel(
        functools.partial(_edge_body, nchunks),
        out_type=[jax.ShapeDtypeStruct((NC, NACC, D), jnp.float32),
                  jax.ShapeDtypeStruct((NC * NACC,), jnp.float32)],
        mesh=mesh,
        scratch_types=(
            [pltpu.VMEM_SHARED((NACC, D), jnp.float32),
             pltpu.VMEM_SHARED((NACC,), jnp.float32)]
            + 2 * [pltpu.VMEM((CHUNK,), jnp.int32),
                   pltpu.VMEM((CHUNK,), jnp.int32),
                   pltpu.VMEM((CHUNK, D), jnp.float32),
                   pltpu.VMEM((CHUNK, D), jnp.float32),
                   pltpu.VMEM((CHUNK,), jnp.float32)]
            + [pltpu.VMEM((D,), jnp.float32)]
            + 8 * [pltpu.SemaphoreType.DMA]
        ),
        compiler_params=cp,
    )
    numer, denom_flat = kfn(xl, xr, att, src, dst)
    return numer, denom_flat.reshape(NC, NACC)


def _pre_body(x_ref, wl_ref, wr_ref, xl_ref, xr_ref):
    x = x_ref[...]
    xl_ref[...] = lax.dot_general(x, wl_ref[...], (((1,), (0,)), ((), ())),
                                  precision=_HI)
    xr_ref[...] = lax.dot_general(x, wr_ref[...], (((1,), (0,)), ((), ())),
                                  precision=_HI)


def _tc_pre(x, wl, wr):
    return pl.pallas_call(
        _pre_body,
        out_shape=[jax.ShapeDtypeStruct((N, D), jnp.float32),
                   jax.ShapeDtypeStruct((N, D), jnp.float32)],
    )(x, wl, wr)


def _combine(numer_ref, denom_ref, b_ref):
    num = numer_ref[0] + numer_ref[1]
    den = denom_ref[0] + denom_ref[1]
    num = num[:N]
    den = jnp.maximum(den[:N], 1e-16)
    return num / den[:, None] + b_ref[...][None, :]


def _mid_body(numer_ref, denom_ref, b_ref, wl_ref, wr_ref, xl_ref, xr_ref):
    h = jnp.maximum(_combine(numer_ref, denom_ref, b_ref), 0.0)
    xl_ref[...] = lax.dot_general(h, wl_ref[...], (((1,), (0,)), ((), ())),
                                  precision=_HI)
    xr_ref[...] = lax.dot_general(h, wr_ref[...], (((1,), (0,)), ((), ())),
                                  precision=_HI)


def _tc_mid(numer, denom, b, wl, wr):
    return pl.pallas_call(
        _mid_body,
        out_shape=[jax.ShapeDtypeStruct((N, D), jnp.float32),
                   jax.ShapeDtypeStruct((N, D), jnp.float32)],
    )(numer, denom, b, wl, wr)


def _post_body(numer_ref, denom_ref, b_ref, w1_ref, b1_ref, w2_ref, b2_ref,
               batch_ref, out_ref):
    h = _combine(numer_ref, denom_ref, b_ref)
    t = lax.dot_general(h, w1_ref[...], (((1,), (0,)), ((), ())),
                        precision=_HI) + b1_ref[...][None, :]
    mu = jnp.mean(t, axis=0, keepdims=True)
    var = jnp.mean((t - mu) ** 2, axis=0, keepdims=True)
    t = (t - mu) / jnp.sqrt(var + 1e-5)
    t = jnp.maximum(t, 0.0)
    y = lax.dot_general(t, w2_ref[...], (((1,), (0,)), ((), ())),
                        precision=_HI)+ b2_ref[...][None, :]
    pt = (batch_ref[...] == lax.broadcasted_iota(jnp.int32, (G, 1), 0)
          ).astype(jnp.float32)
    sums = lax.dot_general(pt, y, (((1,), (0,)), ((), ())), precision=_HI)
    cnt = jnp.sum(pt, axis=1, keepdims=True)
    out_ref[...] = sums / jnp.maximum(cnt, 1.0)


def _tc_post(numer, denom, b, w1, b1, w2, b2, batch2d):
    return pl.pallas_call(
        _post_body,
        out_shape=jax.ShapeDtypeStruct((G, 32), jnp.float32),
    )(numer, denom, b, w1, b1, w2, b2, batch2d)


def kernel(x, edge_index, batch, gat_Wl0, gat_Wr0, gat_att0, gat_b0,
           gat_Wl1, gat_Wr1, gat_att1, gat_b1,
           mlp_W1, mlp_b1, mlp_W2, mlp_b2):
    loops = jnp.arange(N, dtype=jnp.int32)
    src = jnp.concatenate([edge_index[0], loops])
    dst = jnp.concatenate([edge_index[1], loops])
    e_tot = src.shape[0]
    nchunks = 2 * _cdiv(e_tot, NW * CHUNK * 2)
    e_pad = NW * CHUNK * nchunks
    pad = jnp.arange(e_pad - e_tot, dtype=jnp.int32)
    src = jnp.concatenate([src, pad % N])
    dst = jnp.concatenate([dst, N + pad % (NACC - N)])
    batch2d = batch.reshape(1, N)

    xl0, xr0 = _tc_pre(x, gat_Wl0, gat_Wr0)
    num0, den0 = _sc_edge_layer(xl0, xr0, gat_att0, src, dst, nchunks)
    xl1, xr1 = _tc_mid(num0, den0, gat_b0, gat_Wl1, gat_Wr1)
    num1, den1 = _sc_edge_layer(xl1, xr1, gat_att1, src, dst, nchunks)
    return _tc_post(num1, den1, gat_b1, mlp_W1, mlp_b1, mlp_W2, mlp_b2,
                    batch2d)


# speedup vs baseline: 1.0625x; 1.0625x over previous
import dataclasses
import functools

import jax
import jax.numpy as jnp
from jax import lax
from jax.experimental import pallas as pl
from jax.experimental.pallas import tpu as pltpu
from jax.experimental.pallas import tpu_sc as plsc

N = 10000
D = 128
G = 64
NC = 2
NS = 16
LANES = 16
NW = NC * NS
CHUNK = 64
ROWS_PER_TILE = 632
NACC = ROWS_PER_TILE * NS
_ZOFF = tuple(range(0, 513, 64)) + (568,)

_HI = lax.Precision.HIGHEST


def _cdiv(a, b):
    return (a + b - 1) // b


def _edge_body(nchunks, xl_hbm, xr_hbm, att_hbm, src_hbm, dst_hbm,
               numer_hbm, denom_hbm,
               numer_sh, denom_sh,
               xl0, xr0, p0, xl1, xr1, p1,
               srcI00, dstI00, srcI01, dstI01,
               srcI10, dstI10, srcI11, dstI11, att_v,
               gl0, gr0, sn0, sd0, gl1, gr1, sn1, sd1,
               is00, is01, is10, is11):
    c = lax.axis_index("c")
    s = lax.axis_index("s")
    w = c * NS + s
    bufs = ((xl0, xr0, p0, gl0, gr0, sn0, sd0),
            (xl1, xr1, p1, gl1, gr1, sn1, sd1))
    isets = (((srcI00, dstI00, is00), (srcI01, dstI01, is01)),
             ((srcI10, dstI10, is10), (srcI11, dstI11, is11)))

    z16 = jnp.zeros((LANES,), jnp.float32)

    @pl.loop(0, CHUNK)
    def _(r):
        for j in range(D // LANES):
            xl0[r, pl.ds(j * LANES, LANES)] = z16

    @pl.loop(0, CHUNK, step=LANES)
    def _(i):
        p0[pl.ds(i, LANES)] = z16

    r0 = s * ROWS_PER_TILE
    for off in _ZOFF:
        pltpu.sync_copy(xl0, numer_sh.at[pl.ds(r0 + off, CHUNK)])
        pltpu.sync_copy(p0, denom_sh.at[pl.ds(r0 + off, CHUNK)])
    pltpu.sync_copy(att_hbm, att_v)
    plsc.subcore_barrier()

    def idxload(i, iset):
        srcv, dstv, sem = iset
        base = (w * nchunks + i) * CHUNK
        pltpu.async_copy(src_hbm.at[pl.ds(base, CHUNK)], srcv, sem)
        pltpu.async_copy(dst_hbm.at[pl.ds(base, CHUNK)], dstv, sem)

    def wait_idx(i, iset):
        srcv, dstv, sem = iset
        base = (w * nchunks + i) * CHUNK
        pltpu.make_async_copy(src_hbm.at[pl.ds(base, CHUNK)], srcv,
                              sem).wait()
        pltpu.make_async_copy(dst_hbm.at[pl.ds(base, CHUNK)], dstv,
                              sem).wait()

    def start_gather(iset, b):
        srcv, dstv, _ = iset
        xlv, xrv, _, gl, gr, _, _ = bufs[b]
        pltpu.async_copy(xl_hbm.at[srcv], xlv, gl)
        pltpu.async_copy(xr_hbm.at[dstv], xrv, gr)

    def wait_gathers(iset, b):
        srcv, dstv, _ = iset
        xlv, xrv, _, gl, gr, _, _ = bufs[b]
        pltpu.make_async_copy(xl_hbm.at[srcv], xlv, gl).wait()
        pltpu.make_async_copy(xr_hbm.at[dstv], xrv, gr).wait()

    att_regs = [att_v[pl.ds(j * LANES, LANES)] for j in range(D // LANES)]

    def compute(b):
        xlv, xrv, pv = bufs[b][:3]

        @pl.loop(0, CHUNK)
        def _(e):
            acc = jnp.zeros((LANES,), jnp.float32)
            for j in range(D // LANES):
                sl = pl.ds(j * LANES, LANES)
                u = xlv[e, sl] + xrv[e, sl]
                t = jnp.maximum(u, 0.2 * u)
                acc = acc + t * att_regs[j]
            xrv[e, pl.ds(0, LANES)] = acc

        @pl.loop(0, CHUNK, step=LANES)
        def _(eb):
            rows = eb + lax.iota(jnp.int32, LANES)
            ssum = jnp.zeros((LANES,), jnp.float32)
            for col in range(LANES):
                cols = jnp.full((LANES,), col, jnp.int32)
                ssum = ssum + plsc.load_gather(xrv, [rows, cols])
            p16 = jnp.exp(ssum)
            pv[pl.ds(eb, LANES)] = p16
            for l in range(LANES):
                p = p16[l]
                for j in range(D // LANES):
                    sl = pl.ds(j * LANES, LANES)
                    xlv[eb + l, sl] = xlv[eb + l, sl] * p

    def start_scatter(iset, b):
        dstv = iset[1]
        xlv, _, pv, _, _, sn, sd = bufs[b]
        pltpu.async_copy(xlv, numer_sh.at[dstv], sn, add=True)
        pltpu.async_copy(pv, denom_sh.at[dstv], sd, add=True)

    def wait_scatter(iset, b):
        dstv = iset[1]
        xlv, _, pv, _, _, sn, sd = bufs[b]
        pltpu.make_async_copy(xlv, numer_sh.at[dstv], sn).wait()
        pltpu.make_async_copy(pv, denom_sh.at[dstv], sd).wait()

    idxload(0, isets[0][0])
    idxload(1, isets[1][0])
    idxload(2, isets[0][1])
    wait_idx(0, isets[0][0])
    start_gather(isets[0][0], 0)

    @pl.loop(0, nchunks // 4)
    def _(g4):
        for k in range(2):
            i0 = 4 * g4 + 2 * k
            i1 = i0 + 1
            s0 = isets[0][k]
            s1 = isets[1][k]

            if k == 0:
                @pl.when(g4 > 0)
                def _():
                    wait_scatter(isets[1][1], 1)
            else:
                wait_scatter(isets[1][0], 1)

            wait_idx(i1, s1)
            start_gather(s1, 1)

            @pl.when(i1 + 2 < nchunks)
            def _():
                idxload(i1 + 2, isets[1][1 - k])

            wait_gathers(s0, 0)
            compute(0)
            start_scatter(s0, 0)
            wait_gathers(s1, 1)
            compute(1)
            start_scatter(s1, 1)
            wait_scatter(s0, 0)

            @pl.when(i0 + 2 < nchunks)
            def _():
                wait_idx(i0 + 2, isets[0][1 - k])
                start_gather(isets[0][1 - k], 0)

            @pl.when(i0 + 4 < nchunks)
            def _():
                idxload(i0 + 4, isets[0][k])

    wait_scatter(isets[1][1], 1)

    plsc.subcore_barrier()
    pltpu.sync_copy(numer_sh.at[pl.ds(r0, ROWS_PER_TILE)],
                    numer_hbm.at[c, pl.ds(r0, ROWS_PER_TILE)])
    for off in _ZOFF:
        pltpu.sync_copy(denom_sh.at[pl.ds(r0 + off, CHUNK)], p0)
        pltpu.sync_copy(p0, denom_hbm.at[pl.ds(c * NACC + r0 + off, CHUNK)])


def _sc_edge_layer(xl, xr, att, src, dst, nchunks):
    mesh = plsc.VectorSubcoreMesh(core_axis_name="c", subcore_axis_name="s")
    cp = pltpu.CompilerParams()
    if "needs_layout_passes" in pltpu.CompilerParams.__dataclass_fields__:
        cp = dataclasses.replace(cp, needs_layout_passes=False)
    kfn = pl.kernel(
        functools.partial(_edge_body, nchunks),
        out_type=[jax.ShapeDtypeStruct((NC, NACC, D), jnp.float32),
                  jax.ShapeDtypeStruct((NC * NACC,), jnp.float32)],
        mesh=mesh,
        scratch_types=(
            [pltpu.VMEM_SHARED((NACC, D), jnp.float32),
             pltpu.VMEM_SHARED((NACC,), jnp.float32)]
            + 2 * [pltpu.VMEM((CHUNK, D), jnp.float32),
                   pltpu.VMEM((CHUNK, D), jnp.float32),
                   pltpu.VMEM((CHUNK,), jnp.float32)]
            + 4 * [pltpu.VMEM((CHUNK,), jnp.int32),
                   pltpu.VMEM((CHUNK,), jnp.int32)]
            + [pltpu.VMEM((D,), jnp.float32)]
            + 12 * [pltpu.SemaphoreType.DMA]
        ),
        compiler_params=cp,
    )
    numer, denom_flat = kfn(xl, xr, att, src, dst)
    return numer, denom_flat.reshape(NC, NACC)


def _pre_body(x_ref, wl_ref, wr_ref, xl_ref, xr_ref):
    x = x_ref[...]
    xl_ref[...] = lax.dot_general(x, wl_ref[...], (((1,), (0,)), ((), ())),
                                  precision=_HI)
    xr_ref[...] = lax.dot_general(x, wr_ref[...], (((1,), (0,)), ((), ())),
                                  precision=_HI)


def _tc_pre(x, wl, wr):
    return pl.pallas_call(
        _pre_body,
        out_shape=[jax.ShapeDtypeStruct((N, D), jnp.float32),
                   jax.ShapeDtypeStruct((N, D), jnp.float32)],
    )(x, wl, wr)


def _combine(numer_ref, denom_ref, b_ref):
    num = numer_ref[0] + numer_ref[1]
    den = denom_ref[0] + denom_ref[1]
    num = num[:N]
    den = jnp.maximum(den[:N], 1e-16)
    return num / den[:, None] + b_ref[...][None, :]


def _mid_body(numer_ref, denom_ref, b_ref, wl_ref, wr_ref, xl_ref, xr_ref):
    h = jnp.maximum(_combine(numer_ref, denom_ref, b_ref), 0.0)
    xl_ref[...] = lax.dot_general(h, wl_ref[...], (((1,), (0,)), ((), ())),
                                  precision=_HI)
    xr_ref[...] = lax.dot_general(h, wr_ref[...], (((1,), (0,)), ((), ())),
                                  precision=_HI)


def _tc_mid(numer, denom, b, wl, wr):
    return pl.pallas_call(
        _mid_body,
        out_shape=[jax.ShapeDtypeStruct((N, D), jnp.float32),
                   jax.ShapeDtypeStruct((N, D), jnp.float32)],
    )(numer, denom, b, wl, wr)


def _post_body(numer_ref, denom_ref, b_ref, w1_ref, b1_ref, w2_ref, b2_ref,
               batch_ref, out_ref):
    h = _combine(numer_ref, denom_ref, b_ref)
    t = lax.dot_general(h, w1_ref[...], (((1,), (0,)), ((), ())),
                        precision=_HI) + b1_ref[...][None, :]
    mu = jnp.mean(t, axis=0, keepdims=True)
    var = jnp.mean((t - mu) ** 2, axis=0, keepdims=True)
    t = (t - mu) / jnp.sqrt(var + 1e-5)
    t = jnp.maximum(t, 0.0)
    y = lax.dot_general(t, w2_ref[...], (((1,), (0,)), ((), ())),
                        precision=_HI)+ b2_ref[...][None, :]
    pt = (batch_ref[...] == lax.broadcasted_iota(jnp.int32, (G, 1), 0)
          ).astype(jnp.float32)
    sums = lax.dot_general(pt, y, (((1,), (0,)), ((), ())), precision=_HI)
    cnt = jnp.sum(pt, axis=1, keepdims=True)
    out_ref[...] = sums / jnp.maximum(cnt, 1.0)


def _tc_post(numer, denom, b, w1, b1, w2, b2, batch2d):
    return pl.pallas_call(
        _post_body,
        out_shape=jax.ShapeDtypeStruct((G, 32), jnp.float32),
    )(numer, denom, b, w1, b1, w2, b2, batch2d)


def kernel(x, edge_index, batch, gat_Wl0, gat_Wr0, gat_att0, gat_b0,
           gat_Wl1, gat_Wr1, gat_att1, gat_b1,
           mlp_W1, mlp_b1, mlp_W2, mlp_b2):
    loops = jnp.arange(N, dtype=jnp.int32)
    src = jnp.concatenate([edge_index[0], loops])
    dst = jnp.concatenate([edge_index[1], loops])
    e_tot = src.shape[0]
    nchunks = 4 * _cdiv(e_tot, NW * CHUNK * 4)
    e_pad = NW * CHUNK * nchunks
    pad = jnp.arange(e_pad - e_tot, dtype=jnp.int32)
    src = jnp.concatenate([src, pad % N])
    dst = jnp.concatenate([dst, N + pad % (NACC - N)])
    batch2d = batch.reshape(1, N)

    xl0, xr0 = _tc_pre(x, gat_Wl0, gat_Wr0)
    num0, den0 = _sc_edge_layer(xl0, xr0, gat_att0, src, dst, nchunks)
    xl1, xr1 = _tc_mid(num0, den0, gat_b0, gat_Wl1, gat_Wr1)
    num1, den1 = _sc_edge_layer(xl1, xr1, gat_att1, src, dst, nchunks)
    return _tc_post(num1, den1, gat_b1, mlp_W1, mlp_b1, mlp_W2, mlp_b2,
                    batch2d)
